# Initial kernel scaffold; baseline (speedup 1.0000x reference)
import jax
import jax.numpy as jnp
from jax.experimental import pallas as pl


def kernel(x, gate_w, fc1_w, fc1_b, fc2_w, fc2_b):
    raise NotImplementedError("write your pallas kernel here")


# speedup vs baseline: 1.1101x; 1.1101x over previous
import functools
import math

import jax
import jax.numpy as jnp
import numpy as np
from jax.experimental import pallas as pl
from jax.experimental.pallas import tpu as pltpu

_B, _S, _D, _FF, _E, _TOP_K = 32, 1, 1024, 4096, 8, 2
_LB_COEF = 0.01
_FFT = 512
_NJ = _FF // _FFT

_INV_SQRT2 = 1.0 / math.sqrt(2.0)


def _moe_body(x_ref, gw_ref, fc1w_ref, fc1b_ref, fc2w_ref, fc2b_ref,
              out_ref, aux_ref, w_ref):
    e = pl.program_id(0)
    j = pl.program_id(1)

    @pl.when((e == 0) & (j == 0))
    def _gate():
        xv = x_ref[...]
        logits = jax.lax.dot_general(
            xv, gw_ref[...], (((1,), (1,)), ((), ())),
            preferred_element_type=jnp.float32)
        lane = jax.lax.broadcasted_iota(jnp.int32, logits.shape, 1)
        m1 = jnp.max(logits, axis=1, keepdims=True)
        i1 = jnp.min(jnp.where(logits == m1, lane, _E), axis=1, keepdims=True)
        msk1 = lane == i1
        l2 = jnp.where(msk1, -jnp.inf, logits)
        m2 = jnp.max(l2, axis=1, keepdims=True)
        i2 = jnp.min(jnp.where(l2 == m2, lane, _E), axis=1, keepdims=True)
        msk2 = lane == i2
        b = jnp.exp(m2 - m1)
        denom = 1.0 + b
        w1 = 1.0 / denom
        w2 = b / denom
        w_ref[...] = (jnp.where(msk1, w1, 0.0) + jnp.where(msk2, w2, 0.0))
        p = jnp.exp(logits - m1)
        p = p / jnp.sum(p, axis=1, keepdims=True)
        load = jnp.mean(msk1.astype(jnp.float32) + msk2.astype(jnp.float32),
                        axis=0, keepdims=True)
        imp = jnp.mean(p, axis=0, keepdims=True)
        aux_ref[...] = _LB_COEF * _E * jnp.sum(load * imp, axis=1,
                                               keepdims=True)
        out_ref[...] = jnp.zeros_like(out_ref)

    h = jax.lax.dot_general(
        x_ref[...], fc1w_ref[0], (((1,), (1,)), ((), ())),
        preferred_element_type=jnp.float32)
    h = h + fc1b_ref[0, 0, 0]
    h = 0.5 * h * (1.0 + jax.lax.erf(h * _INV_SQRT2))
    part = jax.lax.dot_general(
        h, fc2w_ref[0], (((1,), (1,)), ((), ())),
        preferred_element_type=jnp.float32)

    lane_e = jax.lax.broadcasted_iota(jnp.int32, (_B, _E), 1)
    we = jnp.sum(jnp.where(lane_e == e, w_ref[...], 0.0), axis=1,
                 keepdims=True)
    out_ref[...] += we * part

    @pl.when(j == 0)
    def _bias2():
        out_ref[...] += we * fc2b_ref[0]


@jax.jit
def _moe(x2, gate_w, fc1_w, fc1b_r, fc2_w, fc2b_r):
    out, aux = pl.pallas_call(
        _moe_body,
        grid=(_E, _NJ),
        in_specs=[
            pl.BlockSpec((_B, _D), lambda e, j: (0, 0)),
            pl.BlockSpec((_E, _D), lambda e, j: (0, 0)),
            pl.BlockSpec((1, _FFT, _D), lambda e, j: (e, j, 0)),
            pl.BlockSpec((1, 1, 1, _FFT), lambda e, j: (e, j, 0, 0)),
            pl.BlockSpec((1, _D, _FFT), lambda e, j: (e, 0, j)),
            pl.BlockSpec((1, 1, _D), lambda e, j: (e, 0, 0)),
        ],
        out_specs=[
            pl.BlockSpec((_B, _D), lambda e, j: (0, 0)),
            pl.BlockSpec((1, 1), lambda e, j: (0, 0)),
        ],
        out_shape=[
            jax.ShapeDtypeStruct((_B, _D), jnp.float32),
            jax.ShapeDtypeStruct((1, 1), jnp.float32),
        ],
        scratch_shapes=[pltpu.VMEM((_B, _E), jnp.float32)],
    )(x2, gate_w, fc1_w, fc1b_r, fc2_w, fc2b_r)
    return out, aux


def kernel(x, gate_w, fc1_w, fc1_b, fc2_w, fc2_b):
    x2 = x.reshape(_B * _S, _D)
    fc1b_r = fc1_b.reshape(_E, _NJ, 1, _FFT)
    fc2b_r = fc2_b.reshape(_E, 1, _D)
    out, aux = _moe(x2, gate_w, fc1_w, fc1b_r, fc2_w, fc2b_r)
    return out.reshape(_B, _S, _D), aux.reshape(())


# speedup vs baseline: 1.3381x; 1.2054x over previous
import functools
import math

import jax
import jax.numpy as jnp
import numpy as np
from jax.experimental import pallas as pl
from jax.experimental.pallas import tpu as pltpu

_B, _S, _D, _FF, _E, _TOP_K = 32, 1, 1024, 4096, 8, 2
_LB_COEF = 0.01
_FFT = 1024
_NJ = _FF // _FFT

_INV_SQRT2 = 1.0 / math.sqrt(2.0)


def _moe_body(x_ref, gw_ref, fc1w_ref, fc1b_ref, fc2w_ref, fc2b_ref,
              out_ref, aux_ref, w_ref):
    e = pl.program_id(0)
    j = pl.program_id(1)

    @pl.when((e == 0) & (j == 0))
    def _gate():
        xv = x_ref[...]
        logits = jax.lax.dot_general(
            xv, gw_ref[...], (((1,), (1,)), ((), ())),
            preferred_element_type=jnp.float32)
        lane = jax.lax.broadcasted_iota(jnp.int32, logits.shape, 1)
        m1 = jnp.max(logits, axis=1, keepdims=True)
        i1 = jnp.min(jnp.where(logits == m1, lane, _E), axis=1, keepdims=True)
        msk1 = lane == i1
        l2 = jnp.where(msk1, -jnp.inf, logits)
        m2 = jnp.max(l2, axis=1, keepdims=True)
        i2 = jnp.min(jnp.where(l2 == m2, lane, _E), axis=1, keepdims=True)
        msk2 = lane == i2
        b = jnp.exp(m2 - m1)
        denom = 1.0 + b
        w1 = 1.0 / denom
        w2 = b / denom
        w_ref[...] = (jnp.where(msk1, w1, 0.0) + jnp.where(msk2, w2, 0.0))
        p = jnp.exp(logits - m1)
        p = p / jnp.sum(p, axis=1, keepdims=True)
        load = jnp.mean(msk1.astype(jnp.float32) + msk2.astype(jnp.float32),
                        axis=0, keepdims=True)
        imp = jnp.mean(p, axis=0, keepdims=True)
        aux_ref[...] = _LB_COEF * _E * jnp.sum(load * imp, axis=1,
                                               keepdims=True)
        out_ref[...] = jnp.zeros_like(out_ref)

    h = jax.lax.dot_general(
        x_ref[...], fc1w_ref[0], (((1,), (1,)), ((), ())),
        preferred_element_type=jnp.float32)
    h = h + fc1b_ref[0, 0, 0]
    h = 0.5 * h * (1.0 + jax.lax.erf(h * _INV_SQRT2))
    part = jax.lax.dot_general(
        h, fc2w_ref[0], (((1,), (1,)), ((), ())),
        preferred_element_type=jnp.float32)

    lane_e = jax.lax.broadcasted_iota(jnp.int32, (_B, _E), 1)
    we = jnp.sum(jnp.where(lane_e == e, w_ref[...], 0.0), axis=1,
                 keepdims=True)
    out_ref[...] += we * part

    @pl.when(j == 0)
    def _bias2():
        out_ref[...] += we * fc2b_ref[0]


@jax.jit
def _moe(x2, gate_w, fc1_w, fc1b_r, fc2_w, fc2b_r):
    out, aux = pl.pallas_call(
        _moe_body,
        grid=(_E, _NJ),
        in_specs=[
            pl.BlockSpec((_B, _D), lambda e, j: (0, 0)),
            pl.BlockSpec((_E, _D), lambda e, j: (0, 0)),
            pl.BlockSpec((1, _FFT, _D), lambda e, j: (e, j, 0)),
            pl.BlockSpec((1, 1, 1, _FFT), lambda e, j: (e, j, 0, 0)),
            pl.BlockSpec((1, _D, _FFT), lambda e, j: (e, 0, j)),
            pl.BlockSpec((1, 1, _D), lambda e, j: (e, 0, 0)),
        ],
        out_specs=[
            pl.BlockSpec((_B, _D), lambda e, j: (0, 0)),
            pl.BlockSpec((1, 1), lambda e, j: (0, 0)),
        ],
        out_shape=[
            jax.ShapeDtypeStruct((_B, _D), jnp.float32),
            jax.ShapeDtypeStruct((1, 1), jnp.float32),
        ],
        scratch_shapes=[pltpu.VMEM((_B, _E), jnp.float32)],
    )(x2, gate_w, fc1_w, fc1b_r, fc2_w, fc2b_r)
    return out, aux


def kernel(x, gate_w, fc1_w, fc1_b, fc2_w, fc2_b):
    x2 = x.reshape(_B * _S, _D)
    fc1b_r = fc1_b.reshape(_E, _NJ, 1, _FFT)
    fc2b_r = fc2_b.reshape(_E, 1, _D)
    out, aux = _moe(x2, gate_w, fc1_w, fc1b_r, fc2_w, fc2b_r)
    return out.reshape(_B, _S, _D), aux.reshape(())
